# baseline (speedup 1.0000x reference)
import functools

import jax
import jax.numpy as jnp
from jax.experimental import pallas as pl
from jax.experimental.pallas import tpu as pltpu

DIM = 1024
INTER = 512
E = 8
SHARED_INTER = 1024
T = 2048
TB = 128
HIGH = jax.lax.Precision.HIGHEST


def _silu(a):
    return a * (1.0 / (1.0 + jnp.exp(-a)))


def _routed_body(x_ref, s_ref, w1_ref, w2_ref, w3_ref, o_ref):
    xb = x_ref[...]
    scores = s_ref[...]
    s = scores - jnp.max(scores, axis=-1, keepdims=True)
    es = jnp.exp(s)
    probs = es / jnp.sum(es, axis=-1, keepdims=True)
    eidx = jax.lax.broadcasted_iota(jnp.int32, (TB, E), 1)
    m1 = jnp.max(probs, axis=-1, keepdims=True)
    a1 = jnp.min(jnp.where(probs == m1, eidx, E), axis=-1, keepdims=True)
    sel1 = eidx == a1
    probs_m = jnp.where(sel1, -1.0, probs)
    m2 = jnp.max(probs_m, axis=-1, keepdims=True)
    a2 = jnp.min(jnp.where(probs_m == m2, eidx, E), axis=-1, keepdims=True)
    sel2 = eidx == a2
    w = probs * (sel1 | sel2).astype(probs.dtype)

    acc = jnp.zeros((TB, DIM), jnp.float32)
    for e in range(E):
        a = jax.lax.dot_general(xb, w1_ref[e], (((1,), (1,)), ((), ())),
                                precision=HIGH)
        b = jax.lax.dot_general(xb, w3_ref[e], (((1,), (1,)), ((), ())),
                                precision=HIGH)
        h = _silu(a) * b
        y = jax.lax.dot_general(h, w2_ref[e], (((1,), (1,)), ((), ())),
                                precision=HIGH)
        acc = acc + w[:, e:e + 1] * y
    o_ref[...] = acc


def _shared_body(x_ref, y_ref, ws1_ref, ws2_ref, ws3_ref, o_ref):
    xb = x_ref[...]
    a = jax.lax.dot_general(xb, ws1_ref[...], (((1,), (1,)), ((), ())),
                            precision=HIGH)
    b = jax.lax.dot_general(xb, ws3_ref[...], (((1,), (1,)), ((), ())),
                            precision=HIGH)
    zh = _silu(a) * b
    z = jax.lax.dot_general(zh, ws2_ref[...], (((1,), (1,)), ((), ())),
                            precision=HIGH)
    o_ref[...] = y_ref[...] + z


@jax.jit
def kernel(x, gate_w, w1, w2, w3, ws1, ws2, ws3):
    shape = x.shape
    xt = x.reshape(-1, DIM)
    scores = xt @ gate_w.T
    grid = (T // TB,)
    xspec = pl.BlockSpec((TB, DIM), lambda i: (i, 0))
    sspec = pl.BlockSpec((TB, E), lambda i: (i, 0))
    full = lambda s: pl.BlockSpec(s, lambda i: (0,) * len(s))
    y = pl.pallas_call(
        _routed_body,
        grid=grid,
        in_specs=[xspec, sspec, full((E, INTER, DIM)),
                  full((E, DIM, INTER)), full((E, INTER, DIM))],
        out_specs=xspec,
        out_shape=jax.ShapeDtypeStruct((T, DIM), jnp.float32),
    )(xt, scores, w1, w2, w3)
    out = pl.pallas_call(
        _shared_body,
        grid=grid,
        in_specs=[xspec, xspec, full((SHARED_INTER, DIM)),
                  full((DIM, SHARED_INTER)), full((SHARED_INTER, DIM))],
        out_specs=xspec,
        out_shape=jax.ShapeDtypeStruct((T, DIM), jnp.float32),
    )(xt, y, ws1, ws2, ws3)
    return out.reshape(shape)


# speedup vs baseline: 9.8406x; 9.8406x over previous
import jax
import jax.numpy as jnp
from jax.experimental import pallas as pl

DIM = 1024
INTER = 512
E = 8
SHARED_INTER = 1024
T = 2048
TB = 256


def _silu(a):
    return a * (1.0 / (1.0 + jnp.exp(-a)))


def _dot(a, b):
    return jax.lax.dot_general(a, b, (((1,), (1,)), ((), ())),
                               preferred_element_type=jnp.float32)


def _body(x_ref, s_ref, w1_ref, w2_ref, w3_ref, ws1_ref, ws2_ref, ws3_ref,
          o_ref):
    xb = x_ref[...]
    scores = s_ref[...]
    s = scores - jnp.max(scores, axis=-1, keepdims=True)
    es = jnp.exp(s)
    probs = es / jnp.sum(es, axis=-1, keepdims=True)
    eidx = jax.lax.broadcasted_iota(jnp.int32, (TB, E), 1)
    m1 = jnp.max(probs, axis=-1, keepdims=True)
    a1 = jnp.min(jnp.where(probs == m1, eidx, E), axis=-1, keepdims=True)
    sel1 = eidx == a1
    probs_m = jnp.where(sel1, -1.0, probs)
    m2 = jnp.max(probs_m, axis=-1, keepdims=True)
    a2 = jnp.min(jnp.where(probs_m == m2, eidx, E), axis=-1, keepdims=True)
    sel2 = eidx == a2
    w = probs * (sel1 | sel2).astype(probs.dtype)

    a = _dot(xb, ws1_ref[...])
    b = _dot(xb, ws3_ref[...])
    zh = (_silu(a) * b).astype(jnp.bfloat16)
    acc = _dot(zh, ws2_ref[...])
    for e in range(E):
        a = _dot(xb, w1_ref[e])
        b = _dot(xb, w3_ref[e])
        h = (_silu(a) * b).astype(jnp.bfloat16)
        acc = acc + w[:, e:e + 1] * _dot(h, w2_ref[e])
    o_ref[...] = acc


@jax.jit
def kernel(x, gate_w, w1, w2, w3, ws1, ws2, ws3):
    shape = x.shape
    xt = x.reshape(-1, DIM)
    scores = xt @ gate_w.T
    xb16 = xt.astype(jnp.bfloat16)
    grid = (T // TB,)
    xspec = pl.BlockSpec((TB, DIM), lambda i: (i, 0))
    sspec = pl.BlockSpec((TB, E), lambda i: (i, 0))
    full = lambda s: pl.BlockSpec(s, lambda i: (0,) * len(s))
    out = pl.pallas_call(
        _body,
        grid=grid,
        in_specs=[xspec, sspec, full((E, INTER, DIM)), full((E, DIM, INTER)),
                  full((E, INTER, DIM)), full((SHARED_INTER, DIM)),
                  full((DIM, SHARED_INTER)), full((SHARED_INTER, DIM))],
        out_specs=xspec,
        out_shape=jax.ShapeDtypeStruct((T, DIM), jnp.float32),
    )(xb16, scores, w1.astype(jnp.bfloat16), w2.astype(jnp.bfloat16),
      w3.astype(jnp.bfloat16), ws1.astype(jnp.bfloat16),
      ws2.astype(jnp.bfloat16), ws3.astype(jnp.bfloat16))
    return out.reshape(shape)


# speedup vs baseline: 11.5665x; 1.1754x over previous
import jax
import jax.numpy as jnp
from jax.experimental import pallas as pl

DIM = 1024
INTER = 512
E = 8
SHARED_INTER = 1024
T = 2048
TB = 256


def _silu(a):
    return a * (1.0 / (1.0 + jnp.exp(-a)))


def _dot(a, b):
    return jax.lax.dot_general(a, b, (((1,), (1,)), ((), ())),
                               preferred_element_type=jnp.float32)


def _routed_body(x_ref, s_ref, w1_ref, w2_ref, w3_ref, o_ref):
    xb = x_ref[...]
    scores = s_ref[...]
    s = scores - jnp.max(scores, axis=-1, keepdims=True)
    es = jnp.exp(s)
    probs = es / jnp.sum(es, axis=-1, keepdims=True)
    eidx = jax.lax.broadcasted_iota(jnp.int32, (TB, E), 1)
    m1 = jnp.max(probs, axis=-1, keepdims=True)
    a1 = jnp.min(jnp.where(probs == m1, eidx, E), axis=-1, keepdims=True)
    sel1 = eidx == a1
    probs_m = jnp.where(sel1, -1.0, probs)
    m2 = jnp.max(probs_m, axis=-1, keepdims=True)
    a2 = jnp.min(jnp.where(probs_m == m2, eidx, E), axis=-1, keepdims=True)
    sel2 = eidx == a2
    w = probs * (sel1 | sel2).astype(probs.dtype)

    acc = jnp.zeros((TB, DIM), jnp.float32)
    for e in range(E):
        a = _dot(xb, w1_ref[e])
        b = _dot(xb, w3_ref[e])
        h = _silu(a) * b
        acc = acc + w[:, e:e + 1] * _dot(h, w2_ref[e])
    o_ref[...] = acc


def _shared_body(x_ref, y_ref, ws1_ref, ws2_ref, ws3_ref, o_ref):
    xb = x_ref[...]
    a = _dot(xb, ws1_ref[...])
    b = _dot(xb, ws3_ref[...])
    zh = _silu(a) * b
    o_ref[...] = y_ref[...] + _dot(zh, ws2_ref[...])


@jax.jit
def kernel(x, gate_w, w1, w2, w3, ws1, ws2, ws3):
    shape = x.shape
    xt = x.reshape(-1, DIM)
    scores = xt @ gate_w.T
    grid = (T // TB,)
    xspec = pl.BlockSpec((TB, DIM), lambda i: (i, 0))
    sspec = pl.BlockSpec((TB, E), lambda i: (i, 0))
    full = lambda s: pl.BlockSpec(s, lambda i: (0,) * len(s))
    y = pl.pallas_call(
        _routed_body,
        grid=grid,
        in_specs=[xspec, sspec, full((E, INTER, DIM)), full((E, DIM, INTER)),
                  full((E, INTER, DIM))],
        out_specs=xspec,
        out_shape=jax.ShapeDtypeStruct((T, DIM), jnp.float32),
    )(xt, scores, w1, w2, w3)
    out = pl.pallas_call(
        _shared_body,
        grid=grid,
        in_specs=[xspec, xspec, full((SHARED_INTER, DIM)),
                  full((DIM, SHARED_INTER)), full((SHARED_INTER, DIM))],
        out_specs=xspec,
        out_shape=jax.ShapeDtypeStruct((T, DIM), jnp.float32),
    )(xt, y, ws1, ws2, ws3)
    return out.reshape(shape)
